# baseline (speedup 1.0000x reference)
import jax
import jax.numpy as jnp
from jax.experimental import pallas as pl
from jax.experimental.pallas import tpu as pltpu

_BATCH_SZ = 4096
_NODE = 64
_NBLK = 64
_STEPS = _NODE // _NBLK


def _copy_body(idx_ref, in_ref, out_ref):
    out_ref[...] = in_ref[0].reshape(_NBLK * _BATCH_SZ)


def kernel(para, batch_idx):
    pt = jnp.swapaxes(para, 1, 2)
    idx = jnp.asarray(batch_idx, jnp.int32).reshape(1)
    out = pl.pallas_call(
        _copy_body,
        grid_spec=pltpu.PrefetchScalarGridSpec(
            num_scalar_prefetch=1,
            grid=(_STEPS,),
            in_specs=[pl.BlockSpec((1, _NBLK, _BATCH_SZ), lambda i, r: (r[0], i, 0))],
            out_specs=pl.BlockSpec((_NBLK * _BATCH_SZ,), lambda i, r: (i,)),
        ),
        out_shape=jax.ShapeDtypeStruct((_NODE * _BATCH_SZ,), jnp.float32),
    )(idx, pt)
    return jnp.transpose(out.reshape(_NODE, _BATCH_SZ, 1), (1, 0, 2))


# speedup vs baseline: 1.0589x; 1.0589x over previous
import jax
import jax.numpy as jnp
from jax.experimental import pallas as pl
from jax.experimental.pallas import tpu as pltpu

_BATCH_SZ = 4096
_NODE = 64
_NBLK = 32
_STEPS = _NODE // _NBLK


def _copy_body(idx_ref, in_ref, out_ref):
    out_ref[...] = in_ref[0].reshape(_NBLK * _BATCH_SZ)


def kernel(para, batch_idx):
    pt = jnp.swapaxes(para, 1, 2)
    idx = jnp.asarray(batch_idx, jnp.int32).reshape(1)
    out = pl.pallas_call(
        _copy_body,
        grid_spec=pltpu.PrefetchScalarGridSpec(
            num_scalar_prefetch=1,
            grid=(_STEPS,),
            in_specs=[pl.BlockSpec((1, _NBLK, _BATCH_SZ), lambda i, r: (r[0], i, 0))],
            out_specs=pl.BlockSpec((_NBLK * _BATCH_SZ,), lambda i, r: (i,)),
        ),
        out_shape=jax.ShapeDtypeStruct((_NODE * _BATCH_SZ,), jnp.float32),
    )(idx, pt)
    return jnp.transpose(out.reshape(_NODE, _BATCH_SZ, 1), (1, 0, 2))


# speedup vs baseline: 1.0823x; 1.0221x over previous
import jax
import jax.numpy as jnp
from jax.experimental import pallas as pl
from jax.experimental.pallas import tpu as pltpu

_BATCH_SZ = 4096
_NODE = 64
_NBLK = 32
_STEPS = _NODE // _NBLK


def _copy_body(idx_ref, in_ref, out_ref):
    out_ref[...] = in_ref[0].reshape(_NBLK * _BATCH_SZ)


def kernel(para, batch_idx):
    pt = jnp.swapaxes(para, 1, 2)
    idx = jnp.asarray(batch_idx, jnp.int32).reshape(1)
    out = pl.pallas_call(
        _copy_body,
        grid_spec=pltpu.PrefetchScalarGridSpec(
            num_scalar_prefetch=1,
            grid=(_STEPS,),
            in_specs=[pl.BlockSpec((1, _NBLK, _BATCH_SZ), lambda i, r: (r[0], i, 0))],
            out_specs=pl.BlockSpec((_NBLK * _BATCH_SZ,), lambda i, r: (i,)),
        ),
        out_shape=jax.ShapeDtypeStruct((_NODE * _BATCH_SZ,), jnp.float32),
        compiler_params=pltpu.CompilerParams(dimension_semantics=("arbitrary",)),
    )(idx, pt)
    return jnp.transpose(out.reshape(_NODE, _BATCH_SZ, 1), (1, 0, 2))
